# baseline (speedup 1.0000x reference)
import functools

import jax
import jax.numpy as jnp
from jax import lax
from jax.experimental import pallas as pl
from jax.experimental.pallas import tpu as pltpu
from jax.experimental.pallas import tpu_sc as plsc

N_NODES = 10000
N_EDGES = 160000
HEADS = 4

_NS = 16
_NC = 2
_NW = _NS * _NC
_B = 80
_W = 128
_EPT = 5120
_EPAD = _EPT * _NW
_NB = _EPT // _B
_NP = 10240
_RPT = _NP // _NS


@functools.partial(jax.jit, static_argnames=("num_chunks",))
def _sc_weighted_scatter(table, walpha, src, dst, num_chunks):
    K = num_chunks
    zeros = jnp.zeros((_RPT, _W), jnp.float32)

    mesh = plsc.VectorSubcoreMesh(core_axis_name="c", subcore_axis_name="s")

    @functools.partial(
        pl.kernel,
        out_type=jax.ShapeDtypeStruct((_NC, K, _NP, _W), jnp.float32),
        mesh=mesh,
        scratch_types=[
            pltpu.VMEM_SHARED((_NP, _W), jnp.float32),
            pltpu.VMEM((_EPT,), jnp.int32),
            pltpu.VMEM((_NB, _B), jnp.int32),
            pltpu.VMEM((_EPT,), jnp.float32),
            pltpu.VMEM((_B,), jnp.int32),
            pltpu.VMEM((_B,), jnp.int32),
            pltpu.VMEM((_B, _W), jnp.float32),
            pltpu.VMEM((_B, _W), jnp.float32),
            pltpu.SemaphoreType.DMA,
            pltpu.SemaphoreType.DMA,
            pltpu.SemaphoreType.DMA,
            pltpu.SemaphoreType.DMA,
        ],
    )
    def scatter_kernel(table_hbm, walpha_hbm, src_hbm, dst_hbm, zeros_hbm,
                       out_hbm, acc, srcb, didxb, wchunk,
                       idx0, idx1, rows0, rows1, sem0, sem1, ssem0, ssem1):
        c = lax.axis_index("c")
        s = lax.axis_index("s")
        wid = c * _NS + s
        e_base = wid * _EPT

        pltpu.sync_copy(src_hbm.at[pl.ds(e_base, _EPT)], srcb)
        pltpu.sync_copy(dst_hbm.at[pl.ds(wid * _NB, _NB)], didxb)

        def rebase(j, idx_ref, base):
            for t in range(_B // 16):
                sl = pl.ds(t * 16, 16)
                idx_ref[sl] = srcb[pl.ds(j * _B + t * 16, 16)] + base

        def weight_rows(j, rows_ref):
            for g in range(_B // 16):
                wvec = wchunk[pl.ds(j * _B + g * 16, 16)]
                for l in range(16):
                    wv = jnp.full((16,), wvec[l], jnp.float32)
                    i = g * 16 + l
                    for r in range(_W // 16):
                        sl = pl.ds(r * 16, 16)
                        rows_ref[i, sl] = rows_ref[i, sl] * wv

        @pl.loop(0, K)
        def chunk_loop(k):
            base = k * N_NODES
            pltpu.sync_copy(zeros_hbm, acc.at[pl.ds(s * _RPT, _RPT)])
            pltpu.sync_copy(walpha_hbm.at[pl.ds(k * _EPAD + e_base, _EPT)],
                            wchunk)
            plsc.subcore_barrier()

            rebase(0, idx0, base)
            pltpu.async_copy(table_hbm.at[idx0], rows0, sem0)
            rebase(1, idx1, base)
            pltpu.async_copy(table_hbm.at[idx1], rows1, sem1)

            @pl.loop(0, _NB, step=2)
            def batch_loop(j):
                pltpu.make_async_copy(table_hbm.at[idx0], rows0, sem0).wait()
                weight_rows(j, rows0)
                pltpu.async_copy(rows0, acc.at[didxb.at[j]], ssem0, add=True)
                pltpu.make_async_copy(table_hbm.at[idx1], rows1, sem1).wait()
                weight_rows(j + 1, rows1)
                pltpu.async_copy(rows1, acc.at[didxb.at[j + 1]], ssem1,
                                 add=True)

                @pl.when(j + 2 < _NB)
                def _():
                    pltpu.make_async_copy(rows0, acc.at[didxb.at[j]],
                                          ssem0).wait()
                    rebase(j + 2, idx0, base)
                    pltpu.async_copy(table_hbm.at[idx0], rows0, sem0)
                    pltpu.make_async_copy(rows1, acc.at[didxb.at[j + 1]],
                                          ssem1).wait()
                    rebase(j + 3, idx1, base)
                    pltpu.async_copy(table_hbm.at[idx1], rows1, sem1)

                @pl.when(j + 2 >= _NB)
                def _():
                    pltpu.make_async_copy(rows0, acc.at[didxb.at[j]],
                                          ssem0).wait()
                    pltpu.make_async_copy(rows1, acc.at[didxb.at[j + 1]],
                                          ssem1).wait()

            plsc.subcore_barrier()
            pltpu.sync_copy(acc.at[pl.ds(s * _RPT, _RPT)],
                            out_hbm.at[c, k, pl.ds(s * _RPT, _RPT)])
            plsc.subcore_barrier()

    return scatter_kernel(table, walpha, src, dst, zeros)


def _gatv2_layer(x, edge_attr, src, dst, Wl, bl, Wr, br, We, att, bias, n_nodes):
    H, C = att.shape
    CK = -(-C // _W)
    Cp = CK * _W
    K = H * CK
    xl = x @ Wl + bl
    xr = x @ Wr + br
    ea = edge_attr @ We
    xl3 = xl.reshape(n_nodes, H, C)
    e = jax.nn.leaky_relu(xl3[src] + xr.reshape(n_nodes, H, C)[dst]
                          + ea.reshape(-1, H, C), negative_slope=0.2)
    logits = jnp.einsum('ehc,hc->eh', e, att)
    m = jax.ops.segment_max(logits, dst, num_segments=n_nodes)
    m = jnp.where(jnp.isfinite(m), m, 0.0)
    p = jnp.exp(logits - m[dst])
    ssum = jax.ops.segment_sum(p, dst, num_segments=n_nodes)
    alpha = p / (ssum[dst] + 1e-16)

    xlp = jnp.pad(xl3, ((0, 0), (0, 0), (0, Cp - C)))
    table = (xlp.reshape(n_nodes, H, CK, _W)
             .transpose(1, 2, 0, 3)
             .reshape(K * n_nodes, _W))
    alphat = jnp.pad(alpha.T, ((0, 0), (0, _EPAD - alpha.shape[0])))
    walpha = jnp.repeat(alphat, CK, axis=0).reshape(-1)
    srcp = jnp.pad(src, (0, _EPAD - src.shape[0]))
    dstp = jnp.pad(dst, (0, _EPAD - dst.shape[0])).reshape(_NW * _NB, _B)
    parts = _sc_weighted_scatter(table, walpha, srcp, dstp, num_chunks=K)
    out_chunks = parts[0] + parts[1]
    out = (out_chunks[:, :n_nodes].reshape(H, CK, n_nodes, _W)
           .transpose(2, 0, 1, 3)
           .reshape(n_nodes, H, Cp)[:, :, :C])
    return out.mean(axis=1) + bias


@jax.jit
def _forward(x, edge_index, edge_attr, params):
    src, dst = edge_index[0], edge_index[1]
    n = x.shape[0]
    h = _gatv2_layer(x, edge_attr, src, dst, *params[0:7], n)
    h = jax.nn.leaky_relu(h, negative_slope=0.01)
    h = _gatv2_layer(h, edge_attr, src, dst, *params[7:14], n)
    h = jax.nn.leaky_relu(h, negative_slope=0.01)
    h = _gatv2_layer(h, edge_attr, src, dst, *params[14:21], n)
    return h


def kernel(x, edge_index, edge_attr, W1l, b1l, W1r, b1r, We1, att1, bias1,
           W2l, b2l, W2r, b2r, We2, att2, bias2, W3l, b3l, W3r, b3r, We3,
           att3, bias3):
    params = (W1l, b1l, W1r, b1r, We1, att1, bias1,
              W2l, b2l, W2r, b2r, We2, att2, bias2,
              W3l, b3l, W3r, b3r, We3, att3, bias3)
    return _forward(x, edge_index, edge_attr, params)


# speedup vs baseline: 1.1732x; 1.1732x over previous
import functools

import jax
import jax.numpy as jnp
from jax import lax
from jax.experimental import pallas as pl
from jax.experimental.pallas import tpu as pltpu
from jax.experimental.pallas import tpu_sc as plsc

N_NODES = 10000
N_EDGES = 160000
HEADS = 4

_NS = 16
_NC = 2
_NW = _NS * _NC
_B = 80
_W = 128
_EPT = 5120
_EPAD = _EPT * _NW
_NB = _EPT // _B
_NP = 10240
_RPT = _NP // _NS
_GB = 8
_GNB = _EPT // _GB
_EB = 256
_NBLK = 1000


def _proj_table(x, Wc, bc):
    N, cin = x.shape
    KL = Wc.shape[1] // 128

    def body(x_ref, w_ref, b_ref, out_ref):
        acc = jnp.dot(x_ref[...], w_ref[...], preferred_element_type=jnp.float32)
        out_ref[...] = acc + b_ref[0:1, :]

    nb = N // _NBLK
    return pl.pallas_call(
        body,
        grid=(nb, KL),
        in_specs=[
            pl.BlockSpec((_NBLK, cin), lambda i, k: (i, 0)),
            pl.BlockSpec((cin, 128), lambda i, k: (0, k)),
            pl.BlockSpec((8, 128), lambda i, k: (0, k)),
        ],
        out_specs=pl.BlockSpec((_NBLK, 128), lambda i, k: (i, k)),
        out_shape=jax.ShapeDtypeStruct((N, KL * 128), jnp.float32),
    )(x, Wc, bc)


def _edge_logits(gxl, gxr, eattr_p, Wep, attp, S):
    E, HCp = gxl.shape

    def body(gxl_ref, gxr_ref, ea_ref, We_ref, att_ref, S_ref, out_ref, mx_ref):
        ea = jnp.dot(ea_ref[...], We_ref[...], preferred_element_type=jnp.float32)
        u = gxl_ref[...] + gxr_ref[...] + ea
        e = jnp.where(u > 0, u, 0.2 * u) * att_ref[...]
        lg = jnp.dot(e, S_ref[...], preferred_element_type=jnp.float32)
        out_ref[...] = lg

        @pl.when(pl.program_id(0) == 0)
        def _():
            mx_ref[...] = jnp.full_like(mx_ref[...], -jnp.inf)

        bm = jnp.max(lg, axis=0, keepdims=True)
        mx_ref[...] = jnp.maximum(mx_ref[...], jnp.broadcast_to(bm, mx_ref.shape))

    return pl.pallas_call(
        body,
        grid=(E // _EB,),
        in_specs=[
            pl.BlockSpec((_EB, HCp), lambda i: (i, 0)),
            pl.BlockSpec((_EB, HCp), lambda i: (i, 0)),
            pl.BlockSpec((_EB, 128), lambda i: (i, 0)),
            pl.BlockSpec((128, HCp), lambda i: (0, 0)),
            pl.BlockSpec((1, HCp), lambda i: (0, 0)),
            pl.BlockSpec((HCp, 128), lambda i: (0, 0)),
        ],
        out_specs=[
            pl.BlockSpec((_EB, 128), lambda i: (i, 0)),
            pl.BlockSpec((8, 128), lambda i: (0, 0)),
        ],
        out_shape=[
            jax.ShapeDtypeStruct((E, 128), jnp.float32),
            jax.ShapeDtypeStruct((8, 128), jnp.float32),
        ],
    )(gxl, gxr, eattr_p, Wep, attp, S)


@functools.partial(jax.jit, static_argnames=("width",))
def _sc_gather(table, idx, width):
    mesh = plsc.VectorSubcoreMesh(core_axis_name="c", subcore_axis_name="s")

    @functools.partial(
        pl.kernel,
        out_type=jax.ShapeDtypeStruct((_EPAD, width), jnp.float32),
        mesh=mesh,
        scratch_types=[
            pltpu.VMEM((_GB,), jnp.int32),
            pltpu.VMEM((_GB,), jnp.int32),
            pltpu.VMEM((_GB, width), jnp.float32),
            pltpu.VMEM((_GB, width), jnp.float32),
            pltpu.SemaphoreType.DMA,
            pltpu.SemaphoreType.DMA,
            pltpu.SemaphoreType.DMA,
            pltpu.SemaphoreType.DMA,
        ],
    )
    def gather_kernel(table_hbm, idx_hbm, out_hbm, idx0, idx1, rows0, rows1,
                      gsem0, gsem1, osem0, osem1):
        c = lax.axis_index("c")
        s = lax.axis_index("s")
        e_base = (c * _NS + s) * _EPT

        def load_idx(j, idx_ref):
            pltpu.sync_copy(idx_hbm.at[pl.ds(e_base + j * _GB, _GB)], idx_ref)

        load_idx(0, idx0)
        pltpu.async_copy(table_hbm.at[idx0], rows0, gsem0)
        load_idx(1, idx1)
        pltpu.async_copy(table_hbm.at[idx1], rows1, gsem1)

        @pl.loop(0, _GNB, step=2)
        def batch_loop(j):
            pltpu.make_async_copy(table_hbm.at[idx0], rows0, gsem0).wait()
            pltpu.async_copy(
                rows0, out_hbm.at[pl.ds(e_base + j * _GB, _GB)], osem0)
            pltpu.make_async_copy(table_hbm.at[idx1], rows1, gsem1).wait()
            pltpu.async_copy(
                rows1, out_hbm.at[pl.ds(e_base + (j + 1) * _GB, _GB)], osem1)

            @pl.when(j + 2 < _GNB)
            def _():
                pltpu.make_async_copy(
                    rows0, out_hbm.at[pl.ds(e_base + j * _GB, _GB)],
                    osem0).wait()
                load_idx(j + 2, idx0)
                pltpu.async_copy(table_hbm.at[idx0], rows0, gsem0)
                pltpu.make_async_copy(
                    rows1, out_hbm.at[pl.ds(e_base + (j + 1) * _GB, _GB)],
                    osem1).wait()
                load_idx(j + 3, idx1)
                pltpu.async_copy(table_hbm.at[idx1], rows1, gsem1)

            @pl.when(j + 2 >= _GNB)
            def _():
                pltpu.make_async_copy(
                    rows0, out_hbm.at[pl.ds(e_base + j * _GB, _GB)],
                    osem0).wait()
                pltpu.make_async_copy(
                    rows1, out_hbm.at[pl.ds(e_base + (j + 1) * _GB, _GB)],
                    osem1).wait()

    return gather_kernel(table, idx)


@functools.partial(jax.jit, static_argnames=("num_chunks",))
def _sc_weighted_scatter(table, walpha, src, dst, num_chunks):
    K = num_chunks
    zeros = jnp.zeros((_RPT, _W), jnp.float32)

    mesh = plsc.VectorSubcoreMesh(core_axis_name="c", subcore_axis_name="s")

    @functools.partial(
        pl.kernel,
        out_type=jax.ShapeDtypeStruct((_NC, K, _NP, _W), jnp.float32),
        mesh=mesh,
        scratch_types=[
            pltpu.VMEM_SHARED((_NP, _W), jnp.float32),
            pltpu.VMEM((_EPT,), jnp.int32),
            pltpu.VMEM((_NB, _B), jnp.int32),
            pltpu.VMEM((_EPT,), jnp.float32),
            pltpu.VMEM((_B,), jnp.int32),
            pltpu.VMEM((_B,), jnp.int32),
            pltpu.VMEM((_B, _W), jnp.float32),
            pltpu.VMEM((_B, _W), jnp.float32),
            pltpu.SemaphoreType.DMA,
            pltpu.SemaphoreType.DMA,
            pltpu.SemaphoreType.DMA,
            pltpu.SemaphoreType.DMA,
        ],
    )
    def scatter_kernel(table_hbm, walpha_hbm, src_hbm, dst_hbm, zeros_hbm,
                       out_hbm, acc, srcb, didxb, wchunk,
                       idx0, idx1, rows0, rows1, sem0, sem1, ssem0, ssem1):
        c = lax.axis_index("c")
        s = lax.axis_index("s")
        wid = c * _NS + s
        e_base = wid * _EPT

        pltpu.sync_copy(src_hbm.at[pl.ds(e_base, _EPT)], srcb)
        pltpu.sync_copy(dst_hbm.at[pl.ds(wid * _NB, _NB)], didxb)
        for t in range(_EPT // 16):
            sl = pl.ds(t * 16, 16)
            srcb[sl] = srcb[sl] * K

        def rebase(j, idx_ref, k):
            for t in range(_B // 16):
                sl = pl.ds(t * 16, 16)
                idx_ref[sl] = srcb[pl.ds(j * _B + t * 16, 16)] + k

        def weight_rows(j, rows_ref):
            for g in range(_B // 16):
                wvec = wchunk[pl.ds(j * _B + g * 16, 16)]
                for l in range(16):
                    wv = jnp.full((16,), wvec[l], jnp.float32)
                    i = g * 16 + l
                    for r in range(_W // 16):
                        sl = pl.ds(r * 16, 16)
                        rows_ref[i, sl] = rows_ref[i, sl] * wv

        @pl.loop(0, K)
        def chunk_loop(k):
            pltpu.sync_copy(zeros_hbm, acc.at[pl.ds(s * _RPT, _RPT)])
            pltpu.sync_copy(walpha_hbm.at[pl.ds(k * _EPAD + e_base, _EPT)],
                            wchunk)
            plsc.subcore_barrier()

            rebase(0, idx0, k)
            pltpu.async_copy(table_hbm.at[idx0], rows0, sem0)
            rebase(1, idx1, k)
            pltpu.async_copy(table_hbm.at[idx1], rows1, sem1)

            @pl.loop(0, _NB, step=2)
            def batch_loop(j):
                pltpu.make_async_copy(table_hbm.at[idx0], rows0, sem0).wait()
                weight_rows(j, rows0)
                pltpu.async_copy(rows0, acc.at[didxb.at[j]], ssem0, add=True)
                pltpu.make_async_copy(table_hbm.at[idx1], rows1, sem1).wait()
                weight_rows(j + 1, rows1)
                pltpu.async_copy(rows1, acc.at[didxb.at[j + 1]], ssem1,
                                 add=True)

                @pl.when(j + 2 < _NB)
                def _():
                    pltpu.make_async_copy(rows0, acc.at[didxb.at[j]],
                                          ssem0).wait()
                    rebase(j + 2, idx0, k)
                    pltpu.async_copy(table_hbm.at[idx0], rows0, sem0)
                    pltpu.make_async_copy(rows1, acc.at[didxb.at[j + 1]],
                                          ssem1).wait()
                    rebase(j + 3, idx1, k)
                    pltpu.async_copy(table_hbm.at[idx1], rows1, sem1)

                @pl.when(j + 2 >= _NB)
                def _():
                    pltpu.make_async_copy(rows0, acc.at[didxb.at[j]],
                                          ssem0).wait()
                    pltpu.make_async_copy(rows1, acc.at[didxb.at[j + 1]],
                                          ssem1).wait()

            plsc.subcore_barrier()
            pltpu.sync_copy(acc.at[pl.ds(s * _RPT, _RPT)],
                            out_hbm.at[c, k, pl.ds(s * _RPT, _RPT)])
            plsc.subcore_barrier()

    return scatter_kernel(table, walpha, src, dst, zeros)


def _chunk_weights(Wm, b, H, C, CK, Cp):
    cin = Wm.shape[0]
    W4 = Wm.reshape(cin, H, C)
    W4 = jnp.pad(W4, ((0, 0), (0, 0), (0, Cp - C)))
    bp = jnp.pad(b.reshape(H, C), ((0, 0), (0, Cp - C)))
    return W4.reshape(cin, H * Cp), jnp.broadcast_to(bp.reshape(1, -1), (8, H * Cp))


def _gatv2_layer(x, eattr_p, srcp, dstp, dst2d, Wl, bl, Wr, br, We, att,
                 bias, n_nodes):
    H, C = att.shape
    CK = -(-C // _W)
    Cp = CK * _W
    K = H * CK
    HCp = H * Cp
    E = N_EDGES

    Wlp, blp = _chunk_weights(Wl, bl, H, C, CK, Cp)
    Wrp, brp = _chunk_weights(Wr, br, H, C, CK, Cp)
    Wep = jnp.pad(We.reshape(16, H, C), ((0, 112), (0, 0), (0, Cp - C)))
    Wep = Wep.reshape(128, HCp)
    attp = jnp.pad(att, ((0, 0), (0, Cp - C))).reshape(1, HCp)
    S = (jnp.arange(HCp) // Cp)[:, None] == jnp.arange(128)[None, :]
    S = S.astype(jnp.float32)

    xl = _proj_table(x, Wlp, blp)
    xr = _proj_table(x, Wrp, brp)

    gxl = _sc_gather(xl, srcp, width=HCp)
    gxr = _sc_gather(xr, dstp, width=HCp)

    lgp, _ = _edge_logits(gxl, gxr, eattr_p, Wep, attp, S)
    logits = lgp[:E, :H]

    dst = dstp[:E]
    m = jax.ops.segment_max(logits, dst, num_segments=n_nodes)
    m = jnp.where(jnp.isfinite(m), m, 0.0)
    p = jnp.exp(logits - m[dst])
    ssum = jax.ops.segment_sum(p, dst, num_segments=n_nodes)
    alpha = p / (ssum[dst] + 1e-16)

    alphat = jnp.pad(alpha.T, ((0, 0), (0, _EPAD - E)))
    walpha = jnp.repeat(alphat, CK, axis=0).reshape(-1)
    parts = _sc_weighted_scatter(xl.reshape(n_nodes * K, _W), walpha, srcp,
                                 dst2d, num_chunks=K)
    out_chunks = parts[0] + parts[1]
    out = (out_chunks[:, :n_nodes].reshape(H, CK, n_nodes, _W)
           .transpose(2, 0, 1, 3)
           .reshape(n_nodes, H, Cp)[:, :, :C])
    return out.mean(axis=1) + bias


@jax.jit
def _forward(x, edge_index, edge_attr, params):
    src, dst = edge_index[0], edge_index[1]
    n = x.shape[0]
    srcp = jnp.pad(src, (0, _EPAD - N_EDGES))
    dstp = jnp.pad(dst, (0, _EPAD - N_EDGES))
    dst2d = dstp.reshape(_NW * _NB, _B)
    eattr_p = jnp.pad(edge_attr, ((0, _EPAD - N_EDGES), (0, 112)))

    h = _gatv2_layer(x, eattr_p, srcp, dstp, dst2d, *params[0:7], n)
    h = jax.nn.leaky_relu(h, negative_slope=0.01)
    h = _gatv2_layer(h, eattr_p, srcp, dstp, dst2d, *params[7:14], n)
    h = jax.nn.leaky_relu(h, negative_slope=0.01)
    h = _gatv2_layer(h, eattr_p, srcp, dstp, dst2d, *params[14:21], n)
    return h


def kernel(x, edge_index, edge_attr, W1l, b1l, W1r, b1r, We1, att1, bias1,
           W2l, b2l, W2r, b2r, We2, att2, bias2, W3l, b3l, W3r, b3r, We3,
           att3, bias3):
    params = (W1l, b1l, W1r, b1r, We1, att1, bias1,
              W2l, b2l, W2r, b2r, We2, att2, bias2,
              W3l, b3l, W3r, b3r, We3, att3, bias3)
    return _forward(x, edge_index, edge_attr, params)
